# baseline (speedup 1.0000x reference)
import functools

import jax
import jax.numpy as jnp
from jax import lax
from jax.experimental import pallas as pl
from jax.experimental.pallas import tpu as pltpu
from jax.experimental.pallas import tpu_sc as plsc

_B = 128
_TC_ROWS = 96
_N = 32768
_NC = 2
_NS = 16
_NW = _NC * _NS
_L = 16

_PASSES = ((23, 0xFF, None, 8), (15, 0xFF, 23, 8), (7, 0xFF, 15, 8),
           (0, 0x7F, 7, 7))


def _vtake(v, idx_splat):
    lane = lax.broadcasted_iota(jnp.int32, (_L,), 0)
    picked = jnp.where(lane == idx_splat, v, 0)
    return jnp.broadcast_to(jnp.sum(picked, axis=0), (_L,))


def _suffix_sum(v):
    return lax.rev(jnp.cumsum(lax.rev(v, (0,)), axis=0), (0,))


def _sc_body(row_start, rpw, scores_hbm, k_hbm, thr_hbm, row_v, hist_v,
             carr_v, k_v, thr_v):
    wid = lax.axis_index("s") * _NC + lax.axis_index("c")
    lane = lax.broadcasted_iota(jnp.int32, (_L,), 0)
    zeros16 = jnp.zeros((_L,), jnp.int32)
    ones16 = jnp.ones((_L,), jnp.int32)

    pltpu.sync_copy(k_hbm, k_v)
    kvec = k_v[...]

    thr_acc = zeros16
    for rr in range(rpw):
        row = row_start + wid * rpw + rr
        pltpu.sync_copy(scores_hbm.at[row], row_v)

        hi = zeros16
        cnt_above = zeros16
        for (shift, dmask, hs, width) in _PASSES:
            def zbody(i, c):
                hist_v[pl.ds(i * _L, _L)] = zeros16
                return c

            lax.fori_loop(0, 512, zbody, 0)

            unroll = 8

            def sbody(i, c, shift=shift, dmask=dmask, hs=hs, hi=hi):
                if hs is None:
                    for u in range(unroll):
                        base = (i * unroll + u) * _L
                        e = row_v[pl.ds(base, _L)]
                        bits = lax.bitcast_convert_type(jnp.abs(e), jnp.int32)
                        digit = lax.shift_right_logical(bits, shift) & dmask
                        plsc.addupdate_scatter(
                            hist_v, [digit * (2 * _L) + (u % 2) * _L + lane],
                            ones16)
                else:
                    digits = []
                    masks = []
                    any_in = None
                    for u in range(unroll):
                        base = (i * unroll + u) * _L
                        e = row_v[pl.ds(base, _L)]
                        bits = lax.bitcast_convert_type(jnp.abs(e), jnp.int32)
                        digit = lax.shift_right_logical(bits, shift) & dmask
                        inr = lax.shift_right_logical(bits, hs) == hi
                        digits.append(digit)
                        masks.append(inr)
                        any_in = inr if any_in is None else (any_in | inr)

                    def do_scatter(_):
                        for uu, (dg, mk) in enumerate(zip(digits, masks)):
                            plsc.addupdate_scatter(
                                hist_v,
                                [dg * (2 * _L) + (uu % 2) * _L + lane],
                                ones16, mask=mk)
                        return 0

                    lax.cond(jnp.any(any_in), do_scatter, lambda _: 0, 0)
                return c

            lax.fori_loop(0, _N // _L // unroll, sbody, 0)

            def mbody(j, c):
                acc = zeros16
                for l in range(2 * _L):
                    acc = acc + plsc.load_gather(
                        hist_v, [(j * _L + lane) * (2 * _L) + l])
                carr_v[pl.ds(j * _L, _L)] = acc
                return c

            lax.fori_loop(0, 16, mbody, 0)

            def rbody(l, g):
                tr = plsc.load_gather(carr_v, [lane * _L + l])
                return g + tr

            g_vec = lax.fori_loop(0, _L, rbody, zeros16)
            SG = _suffix_sum(g_vec)
            condg = (cnt_above + SG) >= kvec
            jstar = plsc.all_reduce_population_count(condg) - 1
            cvec = plsc.load_gather(carr_v, [jstar * _L + lane])
            S = _suffix_sum(cvec)
            A = cnt_above + _vtake(SG, jstar) - _vtake(g_vec, jstar)
            cond2 = (A + S) >= kvec
            d_local = plsc.all_reduce_population_count(cond2) - 1
            cnt_above = A + _vtake(S, d_local) - _vtake(cvec, d_local)
            digit = jstar * _L + d_local
            hi = lax.shift_left(hi, width) | digit

        thr_acc = jnp.where(lane == rr, hi, thr_acc)

    thr_v[...] = thr_acc
    pltpu.sync_copy(thr_v, thr_hbm.at[wid])


def _sc_thresholds(scores, k, row_start, nrows):
    rpw = nrows // _NW
    k16 = jnp.full((_L,), k, jnp.int32)
    mesh = plsc.VectorSubcoreMesh(core_axis_name="c", subcore_axis_name="s")
    fn = functools.partial(
        pl.kernel,
        mesh=mesh,
        compiler_params=pltpu.CompilerParams(needs_layout_passes=False),
        out_type=jax.ShapeDtypeStruct((_NW, _L), jnp.int32),
        scratch_types=[
            pltpu.VMEM((_N,), jnp.float32),
            pltpu.VMEM((32 * 256,), jnp.int32),
            pltpu.VMEM((256,), jnp.int32),
            pltpu.VMEM((_L,), jnp.int32),
            pltpu.VMEM((_L,), jnp.int32),
        ],
    )(functools.partial(_sc_body, row_start, rpw))
    thr = fn(scores, k16)
    return thr[:, :rpw].reshape(nrows)


def _tc_thresh_body(k_ref, x_ref, o_ref, bits_ref):
    x = x_ref[...]
    bits_ref[...] = lax.bitcast_convert_type(jnp.abs(x), jnp.int32)
    k = k_ref[0]
    rows = x.shape[0]
    n = x.shape[1]
    chunks = 8
    w = n // chunks

    def body(i, prefix):
        cand = prefix | (jnp.int32(1) << (jnp.int32(30) - i))
        cnt = jnp.zeros((rows, 1), jnp.int32)
        parts = [
            jnp.sum((bits_ref[:, c * w:(c + 1) * w] >= cand).astype(jnp.int32),
                    axis=1, keepdims=True)
            for c in range(chunks)
        ]
        for p in parts:
            cnt = cnt + p
        return jnp.where(cnt >= k, cand, prefix)

    prefix = lax.fori_loop(0, 31, body, jnp.zeros((rows, 1), jnp.int32))
    o_ref[...] = jnp.broadcast_to(prefix, o_ref.shape)


def _tc_thresholds(scores, k):
    rows_per_block = 32
    k_arr = jnp.reshape(jnp.asarray(k, jnp.int32), (1,))
    out = pl.pallas_call(
        _tc_thresh_body,
        grid=(_TC_ROWS // rows_per_block,),
        in_specs=[
            pl.BlockSpec(memory_space=pltpu.SMEM),
            pl.BlockSpec((rows_per_block, _N), lambda i: (i, 0)),
        ],
        out_specs=pl.BlockSpec((rows_per_block, 128), lambda i: (i, 0)),
        out_shape=jax.ShapeDtypeStruct((_TC_ROWS, 128), jnp.int32),
        scratch_shapes=[pltpu.VMEM((rows_per_block, _N), jnp.int32)],
    )(k_arr, scores[:_TC_ROWS])
    return out[:, 0]


def _mask_body(x_ref, t_ref, o_ref):
    x = x_ref[...]
    bits = lax.bitcast_convert_type(jnp.abs(x), jnp.int32)
    o_ref[...] = jnp.where(bits >= t_ref[...], x, 0.0)


def _tail_mask_body(prev_ref, x_ref, t_ref, o_ref):
    del prev_ref
    x = x_ref[...]
    bits = lax.bitcast_convert_type(jnp.abs(x), jnp.int32)
    o_ref[...] = jnp.where(bits >= t_ref[...], x, 0.0)


def kernel(scores, k):
    b, n = scores.shape
    sc_rows = b - _TC_ROWS
    thr_sc = _sc_thresholds(scores, k, _TC_ROWS, sc_rows)
    thr_tc = _tc_thresholds(scores, k)
    rows_per_block = 32
    partial = pl.pallas_call(
        _mask_body,
        grid=(_TC_ROWS // rows_per_block,),
        in_specs=[
            pl.BlockSpec((rows_per_block, n), lambda i: (i, 0)),
            pl.BlockSpec((rows_per_block, 1), lambda i: (i, 0)),
        ],
        out_specs=pl.BlockSpec((rows_per_block, n), lambda i: (i, 0)),
        out_shape=jax.ShapeDtypeStruct((b, n), scores.dtype),
    )(scores, thr_tc[:, None])
    tail_block = _TC_ROWS // rows_per_block
    return pl.pallas_call(
        _tail_mask_body,
        grid=(sc_rows // rows_per_block,),
        in_specs=[
            pl.BlockSpec((rows_per_block, n),
                         lambda i: (i + tail_block, 0)),
            pl.BlockSpec((rows_per_block, n),
                         lambda i: (i + tail_block, 0)),
            pl.BlockSpec((rows_per_block, 1), lambda i: (i, 0)),
        ],
        out_specs=pl.BlockSpec((rows_per_block, n),
                               lambda i: (i + tail_block, 0)),
        out_shape=jax.ShapeDtypeStruct((b, n), scores.dtype),
        input_output_aliases={0: 0},
    )(partial, scores, thr_sc[:, None])


# speedup vs baseline: 1.1128x; 1.1128x over previous
import functools

import jax
import jax.numpy as jnp
from jax import lax
from jax.experimental import pallas as pl
from jax.experimental.pallas import tpu as pltpu
from jax.experimental.pallas import tpu_sc as plsc

_B = 128
_TC_ROWS = 96
_N = 32768
_NC = 2
_NS = 16
_NW = _NC * _NS
_L = 16

_PASSES = ((23, 0xFF, None, 8), (15, 0xFF, 23, 8), (7, 0xFF, 15, 8),
           (0, 0x7F, 7, 7))


def _vtake(v, idx_splat):
    lane = lax.broadcasted_iota(jnp.int32, (_L,), 0)
    picked = jnp.where(lane == idx_splat, v, 0)
    return jnp.broadcast_to(jnp.sum(picked, axis=0), (_L,))


def _suffix_sum(v):
    return lax.rev(jnp.cumsum(lax.rev(v, (0,)), axis=0), (0,))


def _sc_body(row_start, rpw, scores_hbm, k_hbm, thr_hbm, row_v, hist_v,
             carr_v, k_v, thr_v):
    wid = lax.axis_index("s") * _NC + lax.axis_index("c")
    lane = lax.broadcasted_iota(jnp.int32, (_L,), 0)
    zeros16 = jnp.zeros((_L,), jnp.int32)
    ones16 = jnp.ones((_L,), jnp.int32)

    pltpu.sync_copy(k_hbm, k_v)
    kvec = k_v[...]

    thr_acc = zeros16
    for rr in range(rpw):
        row = row_start + wid * rpw + rr
        pltpu.sync_copy(scores_hbm.at[row], row_v)

        hi = zeros16
        cnt_above = zeros16
        for (shift, dmask, hs, width) in _PASSES:
            def zbody(i, c):
                hist_v[pl.ds(i * _L, _L)] = zeros16
                return c

            lax.fori_loop(0, 256, zbody, 0)

            unroll = 8

            def sbody(i, c, shift=shift, dmask=dmask, hs=hs, hi=hi):
                if hs is None:
                    for u in range(unroll):
                        base = (i * unroll + u) * _L
                        e = row_v[pl.ds(base, _L)]
                        bits = lax.bitcast_convert_type(jnp.abs(e), jnp.int32)
                        digit = lax.shift_right_logical(bits, shift) & dmask
                        plsc.addupdate_scatter(
                            hist_v, [digit * _L + lane], ones16)
                else:
                    digits = []
                    masks = []
                    any_in = None
                    for u in range(unroll):
                        base = (i * unroll + u) * _L
                        e = row_v[pl.ds(base, _L)]
                        bits = lax.bitcast_convert_type(jnp.abs(e), jnp.int32)
                        digit = lax.shift_right_logical(bits, shift) & dmask
                        inr = lax.shift_right_logical(bits, hs) == hi
                        digits.append(digit)
                        masks.append(inr)
                        any_in = inr if any_in is None else (any_in | inr)

                    def do_scatter(_):
                        for dg, mk in zip(digits, masks):
                            plsc.addupdate_scatter(
                                hist_v, [dg * _L + lane], ones16, mask=mk)
                        return 0

                    lax.cond(jnp.any(any_in), do_scatter, lambda _: 0, 0)
                return c

            lax.fori_loop(0, _N // _L // unroll, sbody, 0)

            def mbody(j, c):
                acc = zeros16
                for l in range(_L):
                    acc = acc + plsc.load_gather(
                        hist_v, [(j * _L + lane) * _L + l])
                carr_v[pl.ds(j * _L, _L)] = acc
                return c

            lax.fori_loop(0, 16, mbody, 0)

            def rbody(l, g):
                tr = plsc.load_gather(carr_v, [lane * _L + l])
                return g + tr

            g_vec = lax.fori_loop(0, _L, rbody, zeros16)
            SG = _suffix_sum(g_vec)
            condg = (cnt_above + SG) >= kvec
            jstar = plsc.all_reduce_population_count(condg) - 1
            cvec = plsc.load_gather(carr_v, [jstar * _L + lane])
            S = _suffix_sum(cvec)
            A = cnt_above + _vtake(SG, jstar) - _vtake(g_vec, jstar)
            cond2 = (A + S) >= kvec
            d_local = plsc.all_reduce_population_count(cond2) - 1
            cnt_above = A + _vtake(S, d_local) - _vtake(cvec, d_local)
            digit = jstar * _L + d_local
            hi = lax.shift_left(hi, width) | digit

        thr_acc = jnp.where(lane == rr, hi, thr_acc)

    thr_v[...] = thr_acc
    pltpu.sync_copy(thr_v, thr_hbm.at[wid])


def _sc_thresholds(scores, k, row_start, nrows):
    rpw = nrows // _NW
    k16 = jnp.full((_L,), k, jnp.int32)
    mesh = plsc.VectorSubcoreMesh(core_axis_name="c", subcore_axis_name="s")
    fn = functools.partial(
        pl.kernel,
        mesh=mesh,
        compiler_params=pltpu.CompilerParams(needs_layout_passes=False),
        out_type=jax.ShapeDtypeStruct((_NW, _L), jnp.int32),
        scratch_types=[
            pltpu.VMEM((_N,), jnp.float32),
            pltpu.VMEM((16 * 256,), jnp.int32),
            pltpu.VMEM((256,), jnp.int32),
            pltpu.VMEM((_L,), jnp.int32),
            pltpu.VMEM((_L,), jnp.int32),
        ],
    )(functools.partial(_sc_body, row_start, rpw))
    thr = fn(scores, k16)
    return thr[:, :rpw].reshape(nrows)


def _tc_thresh_body(k_ref, x_ref, o_ref, bits_ref):
    x = x_ref[...]
    bits_ref[...] = lax.bitcast_convert_type(jnp.abs(x), jnp.int32)
    k = k_ref[0]
    rows = x.shape[0]
    n = x.shape[1]
    chunks = 8
    w = n // chunks

    def body(i, prefix):
        cand = prefix | (jnp.int32(1) << (jnp.int32(30) - i))
        cnt = jnp.zeros((rows, 1), jnp.int32)
        parts = [
            jnp.sum((bits_ref[:, c * w:(c + 1) * w] >= cand).astype(jnp.int32),
                    axis=1, keepdims=True)
            for c in range(chunks)
        ]
        for p in parts:
            cnt = cnt + p
        return jnp.where(cnt >= k, cand, prefix)

    prefix = lax.fori_loop(0, 31, body, jnp.zeros((rows, 1), jnp.int32))
    o_ref[...] = jnp.broadcast_to(prefix, o_ref.shape)


def _tc_thresholds(scores, k):
    rows_per_block = 32
    k_arr = jnp.reshape(jnp.asarray(k, jnp.int32), (1,))
    out = pl.pallas_call(
        _tc_thresh_body,
        grid=(_TC_ROWS // rows_per_block,),
        in_specs=[
            pl.BlockSpec(memory_space=pltpu.SMEM),
            pl.BlockSpec((rows_per_block, _N), lambda i: (i, 0)),
        ],
        out_specs=pl.BlockSpec((rows_per_block, 128), lambda i: (i, 0)),
        out_shape=jax.ShapeDtypeStruct((_TC_ROWS, 128), jnp.int32),
        scratch_shapes=[pltpu.VMEM((rows_per_block, _N), jnp.int32)],
    )(k_arr, scores[:_TC_ROWS])
    return out[:, 0]


def _mask_body(x_ref, t_ref, o_ref):
    x = x_ref[...]
    bits = lax.bitcast_convert_type(jnp.abs(x), jnp.int32)
    o_ref[...] = jnp.where(bits >= t_ref[...], x, 0.0)


def kernel(scores, k):
    b, n = scores.shape
    thr_sc = _sc_thresholds(scores, k, _TC_ROWS, b - _TC_ROWS)
    thr_tc = _tc_thresholds(scores, k)
    thr = jnp.concatenate([thr_tc, thr_sc])[:, None]
    rows_per_block = 64
    return pl.pallas_call(
        _mask_body,
        grid=(b // rows_per_block,),
        in_specs=[
            pl.BlockSpec((rows_per_block, n), lambda i: (i, 0)),
            pl.BlockSpec((rows_per_block, 1), lambda i: (i, 0)),
        ],
        out_specs=pl.BlockSpec((rows_per_block, n), lambda i: (i, 0)),
        out_shape=jax.ShapeDtypeStruct((b, n), scores.dtype),
    )(scores, thr)
